# baseline (speedup 1.0000x reference)
import functools

import jax
import jax.numpy as jnp
from jax import lax
from jax.experimental import pallas as pl
from jax.experimental.pallas import tpu as pltpu
from jax.experimental.pallas import tpu_sc as plsc

N_NODES = 10000
N_EDGES = 320000
D = 128

NC = 2
NS = 16
NW = NC * NS
CH = 128
CPW = 80
E_PAD = NW * CPW * CH
N_ACC = 10240
RPT = N_ACC // NS


def _make_agg(d: int):
  mesh = plsc.VectorSubcoreMesh(
      core_axis_name="c", subcore_axis_name="s", num_cores=NC, num_subcores=NS)

  @functools.partial(
      pl.kernel,
      out_type=jax.ShapeDtypeStruct((NC, N_ACC, d), jnp.float32),
      mesh=mesh,
      scratch_types=[
          pltpu.VMEM((CPW // 2, CH), jnp.int32),
          pltpu.VMEM((CPW // 2, CH), jnp.int32),
          pltpu.VMEM((CH, d), jnp.float32),
          pltpu.VMEM((CH, d), jnp.float32),
          pltpu.VMEM_SHARED((N_ACC, d), jnp.float32),
          pltpu.SemaphoreType.DMA,
          pltpu.SemaphoreType.DMA,
          pltpu.SemaphoreType.DMA,
          pltpu.SemaphoreType.DMA,
      ],
  )
  def agg(x_hbm, src_hbm, dst_hbm, z_hbm, out_hbm,
          srcv, dstv, rows0, rows1, acc, sem0, sem1, ssem0, ssem1):
    c = lax.axis_index("c")
    s = lax.axis_index("s")
    wid = s * NC + c
    NWH = CPW // 2

    def stage(ph):
      pltpu.sync_copy(src_hbm.at[pl.ds(wid * CPW + ph * NWH, NWH)], srcv)
      pltpu.sync_copy(dst_hbm.at[pl.ds(wid * CPW + ph * NWH, NWH)], dstv)

    def run_phase():
      pltpu.async_copy(x_hbm.at[srcv.at[0]], rows0, sem0)
      pltpu.async_copy(x_hbm.at[srcv.at[1]], rows1, sem1)

      def step(i, carry):
        g0 = 2 * i
        pltpu.make_async_copy(x_hbm.at[srcv.at[g0]], rows0, sem0).wait()
        sc0 = pltpu.async_copy(rows0, acc.at[dstv.at[g0]], ssem0, add=True)
        pltpu.make_async_copy(x_hbm.at[srcv.at[g0 + 1]], rows1, sem1).wait()
        sc1 = pltpu.async_copy(rows1, acc.at[dstv.at[g0 + 1]], ssem1, add=True)
        sc0.wait()

        @pl.when(g0 + 2 < NWH)
        def _prefetch0():
          pltpu.async_copy(x_hbm.at[srcv.at[g0 + 2]], rows0, sem0)

        sc1.wait()

        @pl.when(g0 + 3 < NWH)
        def _prefetch1():
          pltpu.async_copy(x_hbm.at[srcv.at[g0 + 3]], rows1, sem1)

        return carry

      lax.fori_loop(0, NWH // 2, step, 0)

    stage(0)
    r0 = s * RPT
    for k in range(RPT // CH):
      pltpu.sync_copy(z_hbm, acc.at[pl.ds(r0 + k * CH, CH)])
    plsc.subcore_barrier()

    run_phase()
    stage(1)
    run_phase()
    plsc.subcore_barrier()

    for k in range(RPT // CH):
      pltpu.sync_copy(acc.at[pl.ds(r0 + k * CH, CH)],
                      out_hbm.at[c].at[pl.ds(r0 + k * CH, CH)])

  return agg


@functools.cache
def _agg(d: int):
  return _make_agg(d)


N_P3 = N_ACC * 2


@functools.cache
def _agg_pairs():
  mesh = plsc.VectorSubcoreMesh(
      core_axis_name="c", subcore_axis_name="s", num_cores=NC, num_subcores=NS)

  @functools.partial(
      pl.kernel,
      out_type=jax.ShapeDtypeStruct((NW, N_P3), jnp.float32),
      mesh=mesh,
      scratch_types=[
          pltpu.VMEM((CPW, CH), jnp.int32),
          pltpu.VMEM((CPW, CH), jnp.int32),
          pltpu.VMEM((N_P3,), jnp.float32),
          pltpu.VMEM((N_P3,), jnp.float32),
      ],
      compiler_params=pltpu.CompilerParams(needs_layout_passes=False),
  )
  def aggp(z_hbm, src_hbm, dst_hbm, out_hbm, srcv, dstv, zv, accv):
    c = lax.axis_index("c")
    s = lax.axis_index("s")
    wid = s * NC + c
    pltpu.sync_copy(src_hbm.at[pl.ds(wid * CPW, CPW)], srcv)
    pltpu.sync_copy(dst_hbm.at[pl.ds(wid * CPW, CPW)], dstv)
    pltpu.sync_copy(z_hbm, zv)

    zero16 = jnp.zeros((16,), jnp.float32)

    def zstep(i, carry):
      accv[pl.ds(i * 16, 16)] = zero16
      return carry

    lax.fori_loop(0, N_P3 // 16, zstep, 0)

    def step(g, carry):
      for k in range(CH // 16):
        s16 = srcv[g, pl.ds(k * 16, 16)]
        d16 = dstv[g, pl.ds(k * 16, 16)]
        i0 = s16 * 2
        j0 = d16 * 2
        v0 = plsc.load_gather(zv, [i0])
        v1 = plsc.load_gather(zv, [i0 + 1])
        plsc.addupdate_scatter(accv, [j0], v0)
        plsc.addupdate_scatter(accv, [j0 + 1], v1)
      return carry

    lax.fori_loop(0, CPW, step, 0)
    pltpu.sync_copy(accv, out_hbm.at[wid])

  return aggp


def _tc_combine(z3f, b3f, parts):
  def body(z_ref, b_ref, p_ref, o_ref):
    o_ref[...] = z_ref[...] + b_ref[...] + jnp.sum(p_ref[...], axis=0)

  m = N_P3 // 128
  return pl.pallas_call(
      body,
      in_specs=[
          pl.BlockSpec((m, 128), lambda: (0, 0)),
          pl.BlockSpec((m, 128), lambda: (0, 0)),
          pl.BlockSpec((NW, m, 128), lambda: (0, 0, 0)),
      ],
      out_specs=pl.BlockSpec((m, 128), lambda: (0, 0)),
      out_shape=jax.ShapeDtypeStruct((m, 128), jnp.float32),
  )(z3f.reshape(m, 128), b3f.reshape(m, 128), parts.reshape(NW, m, 128))

BM = 1000


def _tc_layer1(x, p, wt, b):
  def body(x_ref, p_ref, wt_ref, b_ref, o_ref):
    h = x_ref[...] + p_ref[0] + p_ref[1]
    h = jnp.dot(h, wt_ref[...], preferred_element_type=jnp.float32)
    o_ref[...] = jnp.maximum(h + b_ref[...], 0.0)

  return pl.pallas_call(
      body,
      grid=(N_NODES // BM,),
      in_specs=[
          pl.BlockSpec((BM, D), lambda i: (i, 0)),
          pl.BlockSpec((NC, BM, D), lambda i: (0, i, 0)),
          pl.BlockSpec((D, D), lambda i: (0, 0)),
          pl.BlockSpec((1, D), lambda i: (0, 0)),
      ],
      out_specs=pl.BlockSpec((BM, D), lambda i: (i, 0)),
      out_shape=jax.ShapeDtypeStruct((N_NODES, D), jnp.float32),
  )(x, p, wt, b)


def _tc_layer2(x, p, wt, b, w3t):
  def body(x_ref, p_ref, wt_ref, b_ref, w3_ref, o_ref):
    h = x_ref[...] + p_ref[0] + p_ref[1]
    h = jnp.dot(h, wt_ref[...], preferred_element_type=jnp.float32)
    h = jnp.maximum(h + b_ref[...], 0.0)
    o_ref[...] = jnp.dot(h, w3_ref[...], preferred_element_type=jnp.float32)

  return pl.pallas_call(
      body,
      grid=(N_NODES // BM,),
      in_specs=[
          pl.BlockSpec((BM, D), lambda i: (i, 0)),
          pl.BlockSpec((NC, BM, D), lambda i: (0, i, 0)),
          pl.BlockSpec((D, D), lambda i: (0, 0)),
          pl.BlockSpec((1, D), lambda i: (0, 0)),
          pl.BlockSpec((D, D), lambda i: (0, 0)),
      ],
      out_specs=pl.BlockSpec((BM, D), lambda i: (i, 0)),
      out_shape=jax.ShapeDtypeStruct((N_NODES, D), jnp.float32),
  )(x, p, wt, b, w3t)


def kernel(x, edge_index, W1, b1, W2, b2, W3, b3):
  e = edge_index.astype(jnp.int32)
  src, dst = e[0], e[1]
  npad = E_PAD - N_EDGES
  pad_i = jnp.arange(npad, dtype=jnp.int32)
  src_p = jnp.concatenate([src, pad_i % N_NODES])
  dst_p = jnp.concatenate([dst, N_NODES + pad_i % (N_ACC - N_NODES)])
  src2d = src_p.reshape(E_PAD // CH, CH)
  dst2d = dst_p.reshape(E_PAD // CH, CH)
  z128 = jnp.zeros((CH, D), jnp.float32)

  p = _agg(D)(x, src2d, dst2d, z128)
  h1 = _tc_layer1(x, p, W1.T, b1.reshape(1, D))
  p = _agg(D)(h1, src2d, dst2d, z128)
  w3t_pad = jnp.zeros((D, D), jnp.float32).at[:, :2].set(W3.T)
  z3pad = _tc_layer2(h1, p, W2.T, b2.reshape(1, D), w3t_pad)
  z3 = z3pad[:, :2]
  z3f = jnp.zeros((N_ACC, 2), jnp.float32).at[:N_NODES].set(z3).reshape(N_P3)
  parts = _agg_pairs()(z3f, src2d, dst2d)
  b3f = jnp.tile(b3, (N_ACC,))
  out = _tc_combine(z3f, b3f, parts)
  return out.reshape(N_ACC, 2)[:N_NODES]


# speedup vs baseline: 1.0415x; 1.0415x over previous
import functools

import jax
import jax.numpy as jnp
from jax import lax
from jax.experimental import pallas as pl
from jax.experimental.pallas import tpu as pltpu
from jax.experimental.pallas import tpu_sc as plsc

N_NODES = 10000
N_EDGES = 320000
D = 128

NC = 2
NS = 16
NW = NC * NS
CH = 128
CPW = 80
E_PAD = NW * CPW * CH
N_ACC = 10240
RPT = N_ACC // NS


def _make_agg(d: int):
  mesh = plsc.VectorSubcoreMesh(
      core_axis_name="c", subcore_axis_name="s", num_cores=NC, num_subcores=NS)

  @functools.partial(
      pl.kernel,
      out_type=jax.ShapeDtypeStruct((NC, N_ACC, d), jnp.float32),
      mesh=mesh,
      scratch_types=[
          pltpu.VMEM((CPW // 2, CH), jnp.int32),
          pltpu.VMEM((CPW // 2, CH), jnp.int32),
          pltpu.VMEM((CH, d), jnp.float32),
          pltpu.VMEM((CH, d), jnp.float32),
          pltpu.VMEM_SHARED((N_ACC, d), jnp.float32),
          pltpu.SemaphoreType.DMA,
          pltpu.SemaphoreType.DMA,
          pltpu.SemaphoreType.DMA,
          pltpu.SemaphoreType.DMA,
      ],
  )
  def agg(x_hbm, src_hbm, dst_hbm, z_hbm, out_hbm,
          srcv, dstv, rows0, rows1, acc, sem0, sem1, ssem0, ssem1):
    c = lax.axis_index("c")
    s = lax.axis_index("s")
    wid = s * NC + c
    NWH = CPW // 2

    def stage(ph):
      pltpu.sync_copy(src_hbm.at[pl.ds(wid * CPW + ph * NWH, NWH)], srcv)
      pltpu.sync_copy(dst_hbm.at[pl.ds(wid * CPW + ph * NWH, NWH)], dstv)

    def run_phase():
      pltpu.async_copy(x_hbm.at[srcv.at[0]], rows0, sem0)
      pltpu.async_copy(x_hbm.at[srcv.at[1]], rows1, sem1)

      def step(i, carry):
        g0 = 2 * i
        pltpu.make_async_copy(x_hbm.at[srcv.at[g0]], rows0, sem0).wait()
        sc0 = pltpu.async_copy(rows0, acc.at[dstv.at[g0]], ssem0, add=True)
        pltpu.make_async_copy(x_hbm.at[srcv.at[g0 + 1]], rows1, sem1).wait()
        sc1 = pltpu.async_copy(rows1, acc.at[dstv.at[g0 + 1]], ssem1, add=True)
        sc0.wait()

        @pl.when(g0 + 2 < NWH)
        def _prefetch0():
          pltpu.async_copy(x_hbm.at[srcv.at[g0 + 2]], rows0, sem0)

        sc1.wait()

        @pl.when(g0 + 3 < NWH)
        def _prefetch1():
          pltpu.async_copy(x_hbm.at[srcv.at[g0 + 3]], rows1, sem1)

        return carry

      lax.fori_loop(0, NWH // 2, step, 0)

    stage(0)
    pltpu.sync_copy(z_hbm, rows0)
    r0 = s * RPT
    for k in range(RPT // CH):
      pltpu.sync_copy(rows0, acc.at[pl.ds(r0 + k * CH, CH)])
    plsc.subcore_barrier()

    run_phase()
    stage(1)
    run_phase()
    plsc.subcore_barrier()

    for k in range(RPT // CH):
      pltpu.sync_copy(acc.at[pl.ds(r0 + k * CH, CH)], rows0)
      pltpu.sync_copy(rows0, out_hbm.at[c].at[pl.ds(r0 + k * CH, CH)])

  return agg


@functools.cache
def _agg(d: int):
  return _make_agg(d)


N_P3 = N_ACC * 2


@functools.cache
def _agg_pairs():
  mesh = plsc.VectorSubcoreMesh(
      core_axis_name="c", subcore_axis_name="s", num_cores=NC, num_subcores=NS)

  @functools.partial(
      pl.kernel,
      out_type=jax.ShapeDtypeStruct((NW, N_P3), jnp.float32),
      mesh=mesh,
      scratch_types=[
          pltpu.VMEM((CPW, CH), jnp.int32),
          pltpu.VMEM((CPW, CH), jnp.int32),
          pltpu.VMEM((N_P3,), jnp.float32),
          pltpu.VMEM((N_P3,), jnp.float32),
      ],
      compiler_params=pltpu.CompilerParams(needs_layout_passes=False),
  )
  def aggp(z_hbm, src_hbm, dst_hbm, out_hbm, srcv, dstv, zv, accv):
    c = lax.axis_index("c")
    s = lax.axis_index("s")
    wid = s * NC + c
    pltpu.sync_copy(src_hbm.at[pl.ds(wid * CPW, CPW)], srcv)
    pltpu.sync_copy(dst_hbm.at[pl.ds(wid * CPW, CPW)], dstv)
    pltpu.sync_copy(z_hbm, zv)

    zero16 = jnp.zeros((16,), jnp.float32)

    def zstep(i, carry):
      accv[pl.ds(i * 16, 16)] = zero16
      return carry

    lax.fori_loop(0, N_P3 // 16, zstep, 0)

    def step(g, carry):
      for k in range(CH // 16):
        s16 = srcv[g, pl.ds(k * 16, 16)]
        d16 = dstv[g, pl.ds(k * 16, 16)]
        i0 = s16 * 2
        j0 = d16 * 2
        v0 = plsc.load_gather(zv, [i0])
        v1 = plsc.load_gather(zv, [i0 + 1])
        plsc.addupdate_scatter(accv, [j0], v0)
        plsc.addupdate_scatter(accv, [j0 + 1], v1)
      return carry

    lax.fori_loop(0, CPW, step, 0)
    pltpu.sync_copy(accv, out_hbm.at[wid])

  return aggp


def _tc_combine(z3f, b3f, parts):
  def body(z_ref, b_ref, p_ref, o_ref):
    o_ref[...] = z_ref[...] + b_ref[...] + jnp.sum(p_ref[...], axis=0)

  m = N_P3 // 128
  return pl.pallas_call(
      body,
      in_specs=[
          pl.BlockSpec((m, 128), lambda: (0, 0)),
          pl.BlockSpec((m, 128), lambda: (0, 0)),
          pl.BlockSpec((NW, m, 128), lambda: (0, 0, 0)),
      ],
      out_specs=pl.BlockSpec((m, 128), lambda: (0, 0)),
      out_shape=jax.ShapeDtypeStruct((m, 128), jnp.float32),
  )(z3f.reshape(m, 128), b3f.reshape(m, 128), parts.reshape(NW, m, 128))

BM = 1000


def _tc_layer1(x, p, wt, b):
  def body(x_ref, p_ref, wt_ref, b_ref, o_ref):
    h = x_ref[...] + p_ref[0] + p_ref[1]
    h = jnp.dot(h, wt_ref[...], preferred_element_type=jnp.float32)
    o_ref[...] = jnp.maximum(h + b_ref[...], 0.0)

  return pl.pallas_call(
      body,
      grid=(N_NODES // BM,),
      in_specs=[
          pl.BlockSpec((BM, D), lambda i: (i, 0)),
          pl.BlockSpec((NC, BM, D), lambda i: (0, i, 0)),
          pl.BlockSpec((D, D), lambda i: (0, 0)),
          pl.BlockSpec((1, D), lambda i: (0, 0)),
      ],
      out_specs=pl.BlockSpec((BM, D), lambda i: (i, 0)),
      out_shape=jax.ShapeDtypeStruct((N_NODES, D), jnp.float32),
  )(x, p, wt, b)


def _tc_layer2(x, p, wt, b, w3t):
  def body(x_ref, p_ref, wt_ref, b_ref, w3_ref, o_ref):
    h = x_ref[...] + p_ref[0] + p_ref[1]
    h = jnp.dot(h, wt_ref[...], preferred_element_type=jnp.float32)
    h = jnp.maximum(h + b_ref[...], 0.0)
    o_ref[...] = jnp.dot(h, w3_ref[...], preferred_element_type=jnp.float32)

  return pl.pallas_call(
      body,
      grid=(N_NODES // BM,),
      in_specs=[
          pl.BlockSpec((BM, D), lambda i: (i, 0)),
          pl.BlockSpec((NC, BM, D), lambda i: (0, i, 0)),
          pl.BlockSpec((D, D), lambda i: (0, 0)),
          pl.BlockSpec((1, D), lambda i: (0, 0)),
          pl.BlockSpec((D, D), lambda i: (0, 0)),
      ],
      out_specs=pl.BlockSpec((BM, D), lambda i: (i, 0)),
      out_shape=jax.ShapeDtypeStruct((N_NODES, D), jnp.float32),
  )(x, p, wt, b, w3t)


def kernel(x, edge_index, W1, b1, W2, b2, W3, b3):
  e = edge_index.astype(jnp.int32)
  src, dst = e[0], e[1]
  npad = E_PAD - N_EDGES
  pad_i = jnp.arange(npad, dtype=jnp.int32)
  src_p = jnp.concatenate([src, pad_i % N_NODES])
  dst_p = jnp.concatenate([dst, N_NODES + pad_i % (N_ACC - N_NODES)])
  src2d = src_p.reshape(E_PAD // CH, CH)
  dst2d = dst_p.reshape(E_PAD // CH, CH)
  z128 = jnp.zeros((CH, D), jnp.float32)

  p = _agg(D)(x, src2d, dst2d, z128)
  h1 = _tc_layer1(x, p, W1.T, b1.reshape(1, D))
  p = _agg(D)(h1, src2d, dst2d, z128)
  w3t_pad = jnp.zeros((D, D), jnp.float32).at[:, :2].set(W3.T)
  z3pad = _tc_layer2(h1, p, W2.T, b2.reshape(1, D), w3t_pad)
  z3 = z3pad[:, :2]
  z3f = jnp.zeros((N_ACC, 2), jnp.float32).at[:N_NODES].set(z3).reshape(N_P3)
  parts = _agg_pairs()(z3f, src2d, dst2d)
  b3f = jnp.tile(b3, (N_ACC,))
  out = _tc_combine(z3f, b3f, parts)
  return out.reshape(N_ACC, 2)[:N_NODES]


# speedup vs baseline: 1.2766x; 1.2257x over previous
import functools

import jax
import jax.numpy as jnp
from jax import lax
from jax.experimental import pallas as pl
from jax.experimental.pallas import tpu as pltpu
from jax.experimental.pallas import tpu_sc as plsc

N_NODES = 10000
N_EDGES = 320000
D = 128

NC = 2
NS = 16
NW = NC * NS
CH = 128
CPW = 80
E_PAD = NW * CPW * CH
N_ACC = 10240
RPT = N_ACC // NS


def _make_agg(d: int):
  mesh = plsc.VectorSubcoreMesh(
      core_axis_name="c", subcore_axis_name="s", num_cores=NC, num_subcores=NS)

  @functools.partial(
      pl.kernel,
      out_type=jax.ShapeDtypeStruct((NC, N_ACC, d), jnp.float32),
      mesh=mesh,
      scratch_types=[
          pltpu.VMEM((CPW // 2, CH), jnp.int32),
          pltpu.VMEM((CPW // 2, CH), jnp.int32),
          pltpu.VMEM((CH, d), jnp.float32),
          pltpu.VMEM((CH, d), jnp.float32),
          pltpu.VMEM_SHARED((N_ACC, d), jnp.float32),
          pltpu.SemaphoreType.DMA,
          pltpu.SemaphoreType.DMA,
          pltpu.SemaphoreType.DMA,
          pltpu.SemaphoreType.DMA,
      ],
  )
  def agg(x_hbm, src_hbm, dst_hbm, z_hbm, out_hbm,
          srcv, dstv, rows0, rows1, acc, sem0, sem1, ssem0, ssem1):
    c = lax.axis_index("c")
    s = lax.axis_index("s")
    wid = s * NC + c
    NWH = CPW // 2

    def stage(ph):
      pltpu.sync_copy(src_hbm.at[pl.ds(wid * CPW + ph * NWH, NWH)], srcv)
      pltpu.sync_copy(dst_hbm.at[pl.ds(wid * CPW + ph * NWH, NWH)], dstv)

    def run_phase():
      pltpu.async_copy(x_hbm.at[srcv.at[0]], rows0, sem0)

      def step(i, carry):
        g0 = 2 * i
        pltpu.async_copy(x_hbm.at[srcv.at[g0 + 1]], rows1, sem1)
        pltpu.make_async_copy(x_hbm.at[srcv.at[g0]], rows0, sem0).wait()
        pltpu.sync_copy(rows0, acc.at[dstv.at[g0]], add=True)

        @pl.when(g0 + 2 < NWH)
        def _prefetch():
          pltpu.async_copy(x_hbm.at[srcv.at[g0 + 2]], rows0, sem0)

        pltpu.make_async_copy(x_hbm.at[srcv.at[g0 + 1]], rows1, sem1).wait()
        pltpu.sync_copy(rows1, acc.at[dstv.at[g0 + 1]], add=True)
        return carry

      lax.fori_loop(0, NWH // 2, step, 0)

    stage(0)
    pltpu.sync_copy(z_hbm, rows0)
    r0 = s * RPT
    for k in range(RPT // CH):
      pltpu.sync_copy(rows0, acc.at[pl.ds(r0 + k * CH, CH)])
    plsc.subcore_barrier()

    run_phase()
    stage(1)
    run_phase()
    plsc.subcore_barrier()

    for k in range(RPT // CH):
      pltpu.sync_copy(acc.at[pl.ds(r0 + k * CH, CH)], rows0)
      pltpu.sync_copy(rows0, out_hbm.at[c].at[pl.ds(r0 + k * CH, CH)])

  return agg


@functools.cache
def _agg(d: int):
  return _make_agg(d)


N_P3 = N_ACC * 2


@functools.cache
def _agg_pairs():
  mesh = plsc.VectorSubcoreMesh(
      core_axis_name="c", subcore_axis_name="s", num_cores=NC, num_subcores=NS)

  @functools.partial(
      pl.kernel,
      out_type=jax.ShapeDtypeStruct((NW, N_P3), jnp.float32),
      mesh=mesh,
      scratch_types=[
          pltpu.VMEM((CPW, CH), jnp.int32),
          pltpu.VMEM((CPW, CH), jnp.int32),
          pltpu.VMEM((N_P3,), jnp.float32),
          pltpu.VMEM((N_P3,), jnp.float32),
      ],
      compiler_params=pltpu.CompilerParams(needs_layout_passes=False),
  )
  def aggp(z_hbm, src_hbm, dst_hbm, out_hbm, srcv, dstv, zv, accv):
    c = lax.axis_index("c")
    s = lax.axis_index("s")
    wid = s * NC + c
    pltpu.sync_copy(src_hbm.at[pl.ds(wid * CPW, CPW)], srcv)
    pltpu.sync_copy(dst_hbm.at[pl.ds(wid * CPW, CPW)], dstv)
    pltpu.sync_copy(z_hbm, zv)

    zero16 = jnp.zeros((16,), jnp.float32)

    def zstep(i, carry):
      accv[pl.ds(i * 16, 16)] = zero16
      return carry

    lax.fori_loop(0, N_P3 // 16, zstep, 0)

    def step(g, carry):
      for k in range(CH // 16):
        s16 = srcv[g, pl.ds(k * 16, 16)]
        d16 = dstv[g, pl.ds(k * 16, 16)]
        i0 = s16 * 2
        j0 = d16 * 2
        v0 = plsc.load_gather(zv, [i0])
        v1 = plsc.load_gather(zv, [i0 + 1])
        plsc.addupdate_scatter(accv, [j0], v0)
        plsc.addupdate_scatter(accv, [j0 + 1], v1)
      return carry

    lax.fori_loop(0, CPW, step, 0)
    pltpu.sync_copy(accv, out_hbm.at[wid])

  return aggp


def _tc_combine(z3f, b3f, parts):
  def body(z_ref, b_ref, p_ref, o_ref):
    o_ref[...] = z_ref[...] + b_ref[...] + jnp.sum(p_ref[...], axis=0)

  m = N_P3 // 128
  return pl.pallas_call(
      body,
      in_specs=[
          pl.BlockSpec((m, 128), lambda: (0, 0)),
          pl.BlockSpec((m, 128), lambda: (0, 0)),
          pl.BlockSpec((NW, m, 128), lambda: (0, 0, 0)),
      ],
      out_specs=pl.BlockSpec((m, 128), lambda: (0, 0)),
      out_shape=jax.ShapeDtypeStruct((m, 128), jnp.float32),
  )(z3f.reshape(m, 128), b3f.reshape(m, 128), parts.reshape(NW, m, 128))

BM = 1000


def _tc_layer1(x, p, wt, b):
  def body(x_ref, p_ref, wt_ref, b_ref, o_ref):
    h = x_ref[...] + p_ref[0] + p_ref[1]
    h = jnp.dot(h, wt_ref[...], preferred_element_type=jnp.float32)
    o_ref[...] = jnp.maximum(h + b_ref[...], 0.0)

  return pl.pallas_call(
      body,
      grid=(N_NODES // BM,),
      in_specs=[
          pl.BlockSpec((BM, D), lambda i: (i, 0)),
          pl.BlockSpec((NC, BM, D), lambda i: (0, i, 0)),
          pl.BlockSpec((D, D), lambda i: (0, 0)),
          pl.BlockSpec((1, D), lambda i: (0, 0)),
      ],
      out_specs=pl.BlockSpec((BM, D), lambda i: (i, 0)),
      out_shape=jax.ShapeDtypeStruct((N_NODES, D), jnp.float32),
  )(x, p, wt, b)


def _tc_layer2(x, p, wt, b, w3t):
  def body(x_ref, p_ref, wt_ref, b_ref, w3_ref, o_ref):
    h = x_ref[...] + p_ref[0] + p_ref[1]
    h = jnp.dot(h, wt_ref[...], preferred_element_type=jnp.float32)
    h = jnp.maximum(h + b_ref[...], 0.0)
    o_ref[...] = jnp.dot(h, w3_ref[...], preferred_element_type=jnp.float32)

  return pl.pallas_call(
      body,
      grid=(N_NODES // BM,),
      in_specs=[
          pl.BlockSpec((BM, D), lambda i: (i, 0)),
          pl.BlockSpec((NC, BM, D), lambda i: (0, i, 0)),
          pl.BlockSpec((D, D), lambda i: (0, 0)),
          pl.BlockSpec((1, D), lambda i: (0, 0)),
          pl.BlockSpec((D, D), lambda i: (0, 0)),
      ],
      out_specs=pl.BlockSpec((BM, D), lambda i: (i, 0)),
      out_shape=jax.ShapeDtypeStruct((N_NODES, D), jnp.float32),
  )(x, p, wt, b, w3t)


def kernel(x, edge_index, W1, b1, W2, b2, W3, b3):
  e = edge_index.astype(jnp.int32)
  src, dst = e[0], e[1]
  npad = E_PAD - N_EDGES
  pad_i = jnp.arange(npad, dtype=jnp.int32)
  src_p = jnp.concatenate([src, pad_i % N_NODES])
  dst_p = jnp.concatenate([dst, N_NODES + pad_i % (N_ACC - N_NODES)])
  src2d = src_p.reshape(E_PAD // CH, CH)
  dst2d = dst_p.reshape(E_PAD // CH, CH)
  z128 = jnp.zeros((CH, D), jnp.float32)

  p = _agg(D)(x, src2d, dst2d, z128)
  h1 = _tc_layer1(x, p, W1.T, b1.reshape(1, D))
  p = _agg(D)(h1, src2d, dst2d, z128)
  w3t_pad = jnp.zeros((D, D), jnp.float32).at[:, :2].set(W3.T)
  z3pad = _tc_layer2(h1, p, W2.T, b2.reshape(1, D), w3t_pad)
  z3 = z3pad[:, :2]
  z3f = jnp.zeros((N_ACC, 2), jnp.float32).at[:N_NODES].set(z3).reshape(N_P3)
  parts = _agg_pairs()(z3f, src2d, dst2d)
  b3f = jnp.tile(b3, (N_ACC,))
  out = _tc_combine(z3f, b3f, parts)
  return out.reshape(N_ACC, 2)[:N_NODES]
